# baseline (speedup 1.0000x reference)
import functools

import jax
import jax.numpy as jnp
from jax import lax
from jax.experimental import pallas as pl
from jax.experimental.pallas import tpu as pltpu
from jax.experimental.pallas import tpu_sc as plsc

BATCH = 16384
EMBED_DIM = 64
NUM_WORKERS = 32
BPW = BATCH // NUM_WORKERS
NUM_CORES = 2


def _bprmf_body(user_hbm, pos_hbm, neg_hbm, uemb_hbm, iemb_hbm,
                pos_out, neg_out,
                idx_u, idx_i, idx_j, u_rows, i_rows, j_rows,
                pos_v, neg_v, sem):
    wid = lax.axis_index("s") * NUM_CORES + lax.axis_index("c")
    base = wid * BPW

    pltpu.sync_copy(user_hbm.at[pl.ds(base, BPW)], idx_u)
    pltpu.sync_copy(pos_hbm.at[pl.ds(base, BPW)], idx_i)
    pltpu.sync_copy(neg_hbm.at[pl.ds(base, BPW)], idx_j)

    cu = pltpu.async_copy(uemb_hbm.at[idx_u], u_rows, sem)
    ci = pltpu.async_copy(iemb_hbm.at[idx_i], i_rows, sem)
    cj = pltpu.async_copy(iemb_hbm.at[idx_j], j_rows, sem)
    cu.wait()
    ci.wait()
    cj.wait()

    lanes = lax.iota(jnp.int32, 16)

    def group(g, carry):
        b0 = g * 16
        p_acc = jnp.zeros((16,), jnp.float32)
        n_acc = jnp.zeros((16,), jnp.float32)
        for b in range(16):
            u0 = u_rows[b0 + b, pl.ds(0, 16)]
            u1 = u_rows[b0 + b, pl.ds(16, 16)]
            u2 = u_rows[b0 + b, pl.ds(32, 16)]
            u3 = u_rows[b0 + b, pl.ds(48, 16)]
            i0 = i_rows[b0 + b, pl.ds(0, 16)]
            i1 = i_rows[b0 + b, pl.ds(16, 16)]
            i2 = i_rows[b0 + b, pl.ds(32, 16)]
            i3 = i_rows[b0 + b, pl.ds(48, 16)]
            j0 = j_rows[b0 + b, pl.ds(0, 16)]
            j1 = j_rows[b0 + b, pl.ds(16, 16)]
            j2 = j_rows[b0 + b, pl.ds(32, 16)]
            j3 = j_rows[b0 + b, pl.ds(48, 16)]
            p = (u0 * i0 + u1 * i1) + (u2 * i2 + u3 * i3)
            n = (u0 * j0 + u1 * j1) + (u2 * j2 + u3 * j3)
            sel = lanes == b
            p_acc = jnp.where(sel, jnp.sum(p), p_acc)
            n_acc = jnp.where(sel, jnp.sum(n), n_acc)
        pos_v[pl.ds(b0, 16)] = p_acc
        neg_v[pl.ds(b0, 16)] = n_acc
        return carry

    lax.fori_loop(0, BPW // 16, group, 0)

    pltpu.sync_copy(pos_v, pos_out.at[pl.ds(base, BPW)])
    pltpu.sync_copy(neg_v, neg_out.at[pl.ds(base, BPW)])


@jax.jit
def kernel(user, pos_item, neg_item, user_emb, item_emb):
    mesh = plsc.VectorSubcoreMesh(core_axis_name="c", subcore_axis_name="s")
    f = pl.kernel(
        _bprmf_body,
        mesh=mesh,
        compiler_params=pltpu.CompilerParams(
            needs_layout_passes=False, use_tc_tiling_on_sc=False),
        out_type=(
            jax.ShapeDtypeStruct((BATCH,), jnp.float32),
            jax.ShapeDtypeStruct((BATCH,), jnp.float32),
        ),
        scratch_types=[
            pltpu.VMEM((BPW,), jnp.int32),
            pltpu.VMEM((BPW,), jnp.int32),
            pltpu.VMEM((BPW,), jnp.int32),
            pltpu.VMEM((BPW, EMBED_DIM), jnp.float32),
            pltpu.VMEM((BPW, EMBED_DIM), jnp.float32),
            pltpu.VMEM((BPW, EMBED_DIM), jnp.float32),
            pltpu.VMEM((BPW,), jnp.float32),
            pltpu.VMEM((BPW,), jnp.float32),
            pltpu.SemaphoreType.DMA,
        ],
    )
    return f(user, pos_item, neg_item, user_emb, item_emb)
